# baseline (speedup 1.0000x reference)
import functools

import jax
import jax.numpy as jnp
from jax import lax
from jax.experimental import pallas as pl
from jax.experimental.pallas import tpu as pltpu
from jax.experimental.pallas import tpu_sc as plsc

D = 64
LANES = 16
TPB = 128
NBUF = 2


def _sc_kernel(l_dim, nbt, tiles_per_w,
               gid_hbm, gval_hbm, table_hbm, w_hbm, b_hbm, out_hbm,
               raw_v, idxp_v, ofs_v, gval_v, gbuf, tbuf, wb_v,
               gsem0, gsem1, osem0, osem1):
    nc = 2
    wid = lax.axis_index("s") * nc + lax.axis_index("c")
    t0 = wid * tiles_per_w
    gsem = [gsem0, gsem1]
    osem = [osem0, osem1]

    pltpu.sync_copy(w_hbm, wb_v.at[0])
    pltpu.sync_copy(b_hbm, wb_v.at[1])
    wv = [wb_v[0, pl.ds(t * LANES, LANES)] for t in range(D // LANES)]
    bv = [wb_v[1, pl.ds(t * LANES, LANES)] for t in range(D // LANES)]
    w_s = [wv[d // LANES][d % LANES] for d in range(D)]
    b_s = [bv[d // LANES][d % LANES] for d in range(D)]
    iota = lax.broadcasted_iota(jnp.int32, (LANES,), 0)
    cvecs = [cg * LANES + iota for cg in range(TPB // LANES)]

    def stage(b, t):
        l = t // nbt
        boff = (t % nbt) * TPB
        pltpu.sync_copy(gid_hbm.at[l, pl.ds(boff, TPB)], raw_v.at[b])
        pltpu.sync_copy(gval_hbm.at[l, pl.ds(boff, TPB)], gval_v.at[b])
        for cg in range(TPB // LANES):
            sl = pl.ds(cg * LANES, LANES)
            rv = raw_v[b, sl]
            idxp_v[b, sl] = rv >> 1
            ofs_v[b, sl] = (rv & 1) << 6
        pltpu.async_copy(table_hbm.at[idxp_v.at[b]], gbuf.at[b], gsem[b])

    def wait_gather(b):
        pltpu.make_async_copy(table_hbm.at[idxp_v.at[b]], gbuf.at[b],
                              gsem[b]).wait()

    def start_wb(b, t):
        l = t // nbt
        bt = t % nbt
        pltpu.async_copy(tbuf.at[b], out_hbm.at[l, :, bt], osem[b])

    def wait_wb(b, t):
        l = t // nbt
        bt = t % nbt
        pltpu.make_async_copy(tbuf.at[b], out_hbm.at[l, :, bt],
                              osem[b]).wait()

    def compute(b):
        ovecs = [ofs_v[b, pl.ds(cg * LANES, LANES)]
                 for cg in range(TPB // LANES)]
        gvecs = [gval_v[b, pl.ds(cg * LANES, LANES)]
                 for cg in range(TPB // LANES)]
        for dt in range(8):
            for di in range(8):
                d = dt * 8 + di
                for cg in range(TPB // LANES):
                    col = plsc.load_gather(gbuf.at[b],
                                           [cvecs[cg], ovecs[cg] + d])
                    tbuf[b, dt, di, pl.ds(cg * LANES, LANES)] = (
                        col + (gvecs[cg] * w_s[d] + b_s[d]))

    for b in range(NBUF):
        stage(b, t0 + b)

    nsuper = tiles_per_w // NBUF

    def super_body(s, _):
        for b in range(NBUF):
            t = t0 + s * NBUF + b
            wait_gather(b)

            @pl.when(s >= 1)
            def _():
                wait_wb(b, t - NBUF)

            compute(b)
            start_wb(b, t)

            @pl.when(s < nsuper - 1)
            def _():
                stage(b, t + NBUF)

        return 0

    lax.fori_loop(0, nsuper, super_body, 0)

    for b in range(NBUF):
        wait_wb(b, t0 + tiles_per_w - NBUF + b)


def kernel(gene_id, gene_val, emb_table, w_val, b_val):
    bsz, l_dim = gene_id.shape
    vocab = emb_table.shape[0]
    nbt = bsz // TPB
    gid_t = jnp.transpose(gene_id).astype(jnp.int32)
    gval_t = jnp.transpose(gene_val)
    table2 = jnp.reshape(emb_table, (vocab // 2, 2 * D))

    info = plsc.get_sparse_core_info()
    nw = info.num_cores * info.num_subcores
    tiles_per_w = (l_dim * nbt) // nw

    mesh = plsc.VectorSubcoreMesh(core_axis_name="c", subcore_axis_name="s")
    run = pl.kernel(
        functools.partial(_sc_kernel, l_dim, nbt, tiles_per_w),
        mesh=mesh,
        compiler_params=pltpu.CompilerParams(
            use_tc_tiling_on_sc=False, needs_layout_passes=False),
        out_type=jax.ShapeDtypeStruct((l_dim, 8, nbt, 8, TPB), jnp.float32),
        scratch_types=[
            pltpu.VMEM((NBUF, TPB), jnp.int32),
            pltpu.VMEM((NBUF, TPB), jnp.int32),
            pltpu.VMEM((NBUF, TPB), jnp.int32),
            pltpu.VMEM((NBUF, TPB), jnp.float32),
            pltpu.VMEM((NBUF, TPB, 2 * D), jnp.float32),
            pltpu.VMEM((NBUF, 8, 8, TPB), jnp.float32),
            pltpu.VMEM((2, D), jnp.float32),
            pltpu.SemaphoreType.DMA,
            pltpu.SemaphoreType.DMA,
            pltpu.SemaphoreType.DMA,
            pltpu.SemaphoreType.DMA,
        ],
    )
    out5 = run(gid_t, gval_t, table2, w_val, b_val)
    return jnp.reshape(jnp.transpose(out5, (2, 4, 0, 1, 3)),
                       (bsz, l_dim, D))


# speedup vs baseline: 1.4264x; 1.4264x over previous
import functools

import jax
import jax.numpy as jnp
from jax import lax
from jax.experimental import pallas as pl
from jax.experimental.pallas import tpu as pltpu
from jax.experimental.pallas import tpu_sc as plsc

D = 64
LANES = 16
TPB = 128
NBUF = 2


def _sc_kernel(l_dim, nbt, tiles_per_w,
               gid_hbm, gval_hbm, table_hbm, w_hbm, b_hbm, out_hbm,
               raw_v, idxp_v, ofs_v, gval_v, gbuf, tbuf, wb_v,
               wbb_v, bbb_v, gsem0, gsem1, osem0, osem1):
    nc = 2
    wid = lax.axis_index("s") * nc + lax.axis_index("c")
    t0 = wid * tiles_per_w
    gsem = [gsem0, gsem1]
    osem = [osem0, osem1]

    pltpu.sync_copy(w_hbm, wb_v.at[0])
    pltpu.sync_copy(b_hbm, wb_v.at[1])
    wv = [wb_v[0, pl.ds(t * LANES, LANES)] for t in range(D // LANES)]
    bv = [wb_v[1, pl.ds(t * LANES, LANES)] for t in range(D // LANES)]
    for d in range(D):
        wbb_v[d] = lax.broadcast(wv[d // LANES][d % LANES], (LANES,))
        bbb_v[d] = lax.broadcast(bv[d // LANES][d % LANES], (LANES,))
    iota = lax.broadcasted_iota(jnp.int32, (LANES,), 0)
    cvecs = [cg * LANES + iota for cg in range(TPB // LANES)]

    def stage(b, t):
        l = t // nbt
        boff = (t % nbt) * TPB
        pltpu.sync_copy(gid_hbm.at[l, pl.ds(boff, TPB)], raw_v.at[b])
        pltpu.sync_copy(gval_hbm.at[l, pl.ds(boff, TPB)], gval_v.at[b])
        for cg in range(TPB // LANES):
            sl = pl.ds(cg * LANES, LANES)
            rv = raw_v[b, sl]
            idxp_v[b, sl] = rv >> 1
            ofs_v[b, sl] = (rv & 1) << 6
        pltpu.async_copy(table_hbm.at[idxp_v.at[b]], gbuf.at[b], gsem[b])

    def wait_gather(b):
        pltpu.make_async_copy(table_hbm.at[idxp_v.at[b]], gbuf.at[b],
                              gsem[b]).wait()

    def start_wb(b, t):
        l = t // nbt
        bt = t % nbt
        pltpu.async_copy(tbuf.at[b], out_hbm.at[l, :, bt], osem[b])

    def wait_wb(b, t):
        l = t // nbt
        bt = t % nbt
        pltpu.make_async_copy(tbuf.at[b], out_hbm.at[l, :, bt],
                              osem[b]).wait()

    def compute(b):
        for cg in range(TPB // LANES):
            sl = pl.ds(cg * LANES, LANES)
            ovec = ofs_v[b, sl]
            gvec = gval_v[b, sl]
            cvec = cvecs[cg]

            @plsc.parallel_loop(0, 8, unroll=2)
            def _(dt):
                for di in range(8):
                    d = dt * 8 + di
                    col = plsc.load_gather(gbuf.at[b], [cvec, ovec + d])
                    tbuf[b, dt, di, sl] = col + (gvec * wbb_v[d] + bbb_v[d])

    for b in range(NBUF):
        stage(b, t0 + b)

    nsuper = tiles_per_w // NBUF

    def super_body(s, _):
        for b in range(NBUF):
            t = t0 + s * NBUF + b
            wait_gather(b)

            @pl.when(s >= 1)
            def _():
                wait_wb(b, t - NBUF)

            compute(b)
            start_wb(b, t)

            @pl.when(s < nsuper - 1)
            def _():
                stage(b, t + NBUF)

        return 0

    lax.fori_loop(0, nsuper, super_body, 0)

    for b in range(NBUF):
        wait_wb(b, t0 + tiles_per_w - NBUF + b)


def kernel(gene_id, gene_val, emb_table, w_val, b_val):
    bsz, l_dim = gene_id.shape
    vocab = emb_table.shape[0]
    nbt = bsz // TPB
    gid_t = jnp.transpose(gene_id).astype(jnp.int32)
    gval_t = jnp.transpose(gene_val)
    table2 = jnp.reshape(emb_table, (vocab // 2, 2 * D))

    info = plsc.get_sparse_core_info()
    nw = info.num_cores * info.num_subcores
    tiles_per_w = (l_dim * nbt) // nw

    mesh = plsc.VectorSubcoreMesh(core_axis_name="c", subcore_axis_name="s")
    run = pl.kernel(
        functools.partial(_sc_kernel, l_dim, nbt, tiles_per_w),
        mesh=mesh,
        compiler_params=pltpu.CompilerParams(
            use_tc_tiling_on_sc=False, needs_layout_passes=False),
        out_type=jax.ShapeDtypeStruct((l_dim, 8, nbt, 8, TPB), jnp.float32),
        scratch_types=[
            pltpu.VMEM((NBUF, TPB), jnp.int32),
            pltpu.VMEM((NBUF, TPB), jnp.int32),
            pltpu.VMEM((NBUF, TPB), jnp.int32),
            pltpu.VMEM((NBUF, TPB), jnp.float32),
            pltpu.VMEM((NBUF, TPB, 2 * D), jnp.float32),
            pltpu.VMEM((NBUF, 8, 8, TPB), jnp.float32),
            pltpu.VMEM((2, D), jnp.float32),
            pltpu.VMEM((D, LANES), jnp.float32),
            pltpu.VMEM((D, LANES), jnp.float32),
            pltpu.SemaphoreType.DMA,
            pltpu.SemaphoreType.DMA,
            pltpu.SemaphoreType.DMA,
            pltpu.SemaphoreType.DMA,
        ],
    )
    out5 = run(gid_t, gval_t, table2, w_val, b_val)
    return jnp.reshape(jnp.transpose(out5, (2, 4, 0, 1, 3)),
                       (bsz, l_dim, D))
